# baseline (speedup 1.0000x reference)
import functools

import jax
import jax.numpy as jnp
from jax import lax
from jax.experimental import pallas as pl
from jax.experimental.pallas import tpu as pltpu
from jax.experimental.pallas import tpu_sc as plsc

B = 1024
IN_DIM = 4096
OUT_DIM = 512
K = 16
DIMS = (2048, 1024, 512)

LANES = 16
CHUNK = 16
DBLK = 256
TBLK = K * DBLK
NBLKS = sum(DIMS) // DBLK
WMASK = -65536
IMASK = 0xFFFF


def _sc_lcn_kernel(x_hbm, tbl_hbm, out_hbm,
                   bufA, bufB, buf2, tbl0, tbl1, sem0, sem1, sem_in):
    info = plsc.get_sparse_core_info()
    nc = info.num_cores
    wid = lax.axis_index("s") * nc + lax.axis_index("c")
    nw = nc * info.num_subcores

    def tbl_dma(blk, buf, sem):
        return pltpu.make_async_copy(
            tbl_hbm.at[pl.ds(blk * TBLK, TBLK)], buf, sem)

    def run_layer(src_views, dst_ref, dst_w, dim, gbase):
        def proc_block(tbl_buf, d0):
            def grp_body(g, _):
                c0 = pl.multiple_of(g * (2 * LANES), 2 * LANES)

                def k_body(k, accs):
                    for h in range(2):
                        ch = c0 + h * LANES
                        word = tbl_buf[pl.ds(k * DBLK + ch, LANES)]
                        kn = word & IMASK
                        wv = plsc.bitcast(word & WMASK, jnp.float32)
                        accs = accs[:h * CHUNK] + tuple(
                            accs[h * CHUNK + b]
                            + wv * plsc.load_gather(src_views[b], [kn])
                            for b in range(CHUNK)
                        ) + accs[(h + 1) * CHUNK:]
                    return accs

                zero = jnp.zeros((LANES,), jnp.float32)
                accs = lax.fori_loop(0, K, k_body, (zero,) * (2 * CHUNK))
                for h in range(2):
                    for b in range(CHUNK):
                        dst_ref[pl.ds(b * dst_w + d0 + c0 + h * LANES,
                                      LANES)] = jnp.maximum(
                            accs[h * CHUNK + b], 0.0)
                return 0

            lax.fori_loop(0, DBLK // (2 * LANES), grp_body, 0)

        def pair_body(i, _):
            g = gbase + 2 * i
            tbl_dma(0, tbl0, sem0).wait()
            proc_block(tbl0, (2 * i) * DBLK)

            @pl.when(g + 2 < NBLKS)
            def _():
                tbl_dma(g + 2, tbl0, sem0).start()

            tbl_dma(0, tbl1, sem1).wait()
            proc_block(tbl1, (2 * i + 1) * DBLK)

            @pl.when(g + 3 < NBLKS)
            def _():
                tbl_dma(g + 3, tbl1, sem1).start()

            return 0

        lax.fori_loop(0, dim // DBLK // 2, pair_body, 0)

    def in_dma(ci):
        row0 = (wid + ci * nw) * CHUNK
        return pltpu.make_async_copy(
            x_hbm.at[pl.ds(row0 * IN_DIM, CHUNK * IN_DIM)], bufA, sem_in)

    nchunks = B // (nw * CHUNK)

    def chunk_body(ci, _):
        row0 = (wid + ci * nw) * CHUNK
        tbl_dma(0, tbl0, sem0).start()
        tbl_dma(1, tbl1, sem1).start()
        in_dma(0).wait()
        run_layer(views(bufA, IN_DIM), bufB, DIMS[0], DIMS[0], 0)

        @pl.when(ci + 1 < nchunks)
        def _():
            in_dma(ci + 1).start()

        run_layer(views(bufB, DIMS[0]), buf2, DIMS[1], DIMS[1], 8)
        run_layer(views(buf2, DIMS[1]), bufB, DIMS[2], DIMS[2], 12)
        pltpu.sync_copy(bufB.at[pl.ds(0, CHUNK * DIMS[2])],
                        out_hbm.at[pl.ds(row0 * DIMS[2], CHUNK * DIMS[2])])
        return 0

    def views(ref, w):
        return [ref.at[pl.ds(b * w, w)] for b in range(CHUNK)]

    in_dma(0).start()
    lax.fori_loop(0, nchunks, chunk_body, 0)


@functools.partial(
    pl.kernel,
    out_type=jax.ShapeDtypeStruct((B * DIMS[2],), jnp.float32),
    mesh=plsc.VectorSubcoreMesh(core_axis_name="c", subcore_axis_name="s"),
    compiler_params=pltpu.CompilerParams(use_tc_tiling_on_sc=False,
                                         needs_layout_passes=False),
    scratch_types=[
        pltpu.VMEM((CHUNK * IN_DIM,), jnp.float32),
        pltpu.VMEM((CHUNK * DIMS[0],), jnp.float32),
        pltpu.VMEM((CHUNK * DIMS[1],), jnp.float32),
        pltpu.VMEM((TBLK,), jnp.int32),
        pltpu.VMEM((TBLK,), jnp.int32),
        pltpu.SemaphoreType.DMA,
        pltpu.SemaphoreType.DMA,
        pltpu.SemaphoreType.DMA,
    ],
)
def _sc_lcn(*refs):
    _sc_lcn_kernel(*refs)


def _pack_table(knn, w):
    wbits = lax.bitcast_convert_type(
        w.astype(jnp.bfloat16), jnp.uint16).astype(jnp.int32) << 16
    word = wbits | knn
    dim = knn.shape[0]
    nblk = dim // DBLK
    return word.T.reshape(K, nblk, DBLK).swapaxes(0, 1).reshape(-1)


def _fc_body(x_ref, wt_ref, b_ref, o_ref):
    o_ref[...] = jnp.dot(x_ref[...], wt_ref[...],
                         preferred_element_type=jnp.float32) + b_ref[...]


def _fc(x3, fc_wt, fc_b2):
    return pl.pallas_call(
        _fc_body,
        out_shape=jax.ShapeDtypeStruct((B, OUT_DIM), jnp.float32),
    )(x3, fc_wt, fc_b2)


def kernel(input, w0, w1, w2, fc_w, fc_b, knn0, knn1, knn2):
    tbl = jnp.concatenate([_pack_table(knn0, w0),
                           _pack_table(knn1, w1),
                           _pack_table(knn2, w2)])
    x3 = _sc_lcn(input.reshape(-1), tbl)
    return _fc(x3.reshape(B, DIMS[2]), fc_w.T,
               jnp.broadcast_to(fc_b, (1, OUT_DIM)))


# speedup vs baseline: 1.0925x; 1.0925x over previous
import functools

import jax
import jax.numpy as jnp
from jax import lax
from jax.experimental import pallas as pl
from jax.experimental.pallas import tpu as pltpu
from jax.experimental.pallas import tpu_sc as plsc

B = 1024
IN_DIM = 4096
OUT_DIM = 512
K = 16
DIMS = (2048, 1024, 512)

LANES = 16
CHUNK = 16
DBLK = 256
TBLK = K * DBLK
NBLKS = sum(DIMS) // DBLK
WMASK = -65536
IMASK = 0xFFFF


def _sc_lcn_kernel(x_hbm, tbl_hbm, out_hbm,
                   bufA, bufB, buf2, tbl0, tbl1, sem0, sem1, sem_in):
    info = plsc.get_sparse_core_info()
    nc = info.num_cores
    wid = lax.axis_index("s") * nc + lax.axis_index("c")
    nw = nc * info.num_subcores

    def tbl_dma(blk, buf, sem):
        return pltpu.make_async_copy(
            tbl_hbm.at[pl.ds(blk * TBLK, TBLK)], buf, sem)

    def run_layer(src_views, dst_ref, dst_w, dim, gbase):
        def proc_block(tbl_buf, d0):
            def grp_body(g, _):
                c0 = pl.multiple_of(g * LANES, LANES)

                def k_body(i, accs):
                    for j in range(2):
                        k = 2 * i + j
                        word = tbl_buf[pl.ds(k * DBLK + c0, LANES)]
                        kn = word & IMASK
                        wv = plsc.bitcast(word & WMASK, jnp.float32)
                        accs = tuple(
                            accs[b]
                            + wv * plsc.load_gather(src_views[b], [kn])
                            for b in range(CHUNK))
                    return accs

                zero = jnp.zeros((LANES,), jnp.float32)
                accs = lax.fori_loop(0, K // 2, k_body, (zero,) * CHUNK)
                for b in range(CHUNK):
                    dst_ref[pl.ds(b * dst_w + d0 + c0, LANES)] = jnp.maximum(
                        accs[b], 0.0)
                return 0

            lax.fori_loop(0, DBLK // LANES, grp_body, 0)

        def pair_body(i, _):
            g = gbase + 2 * i
            tbl_dma(0, tbl0, sem0).wait()
            proc_block(tbl0, (2 * i) * DBLK)

            @pl.when(g + 2 < NBLKS)
            def _():
                tbl_dma(g + 2, tbl0, sem0).start()

            tbl_dma(0, tbl1, sem1).wait()
            proc_block(tbl1, (2 * i + 1) * DBLK)

            @pl.when(g + 3 < NBLKS)
            def _():
                tbl_dma(g + 3, tbl1, sem1).start()

            return 0

        lax.fori_loop(0, dim // DBLK // 2, pair_body, 0)

    def in_dma(ci):
        row0 = (wid + ci * nw) * CHUNK
        return pltpu.make_async_copy(
            x_hbm.at[pl.ds(row0 * IN_DIM, CHUNK * IN_DIM)], bufA, sem_in)

    nchunks = B // (nw * CHUNK)

    def chunk_body(ci, _):
        row0 = (wid + ci * nw) * CHUNK
        tbl_dma(0, tbl0, sem0).start()
        tbl_dma(1, tbl1, sem1).start()
        in_dma(0).wait()
        run_layer(views(bufA, IN_DIM), bufB, DIMS[0], DIMS[0], 0)

        @pl.when(ci + 1 < nchunks)
        def _():
            in_dma(ci + 1).start()

        run_layer(views(bufB, DIMS[0]), buf2, DIMS[1], DIMS[1], 8)
        run_layer(views(buf2, DIMS[1]), bufB, DIMS[2], DIMS[2], 12)
        pltpu.sync_copy(bufB.at[pl.ds(0, CHUNK * DIMS[2])],
                        out_hbm.at[pl.ds(row0 * DIMS[2], CHUNK * DIMS[2])])
        return 0

    def views(ref, w):
        return [ref.at[pl.ds(b * w, w)] for b in range(CHUNK)]

    in_dma(0).start()
    lax.fori_loop(0, nchunks, chunk_body, 0)


@functools.partial(
    pl.kernel,
    out_type=jax.ShapeDtypeStruct((B * DIMS[2],), jnp.float32),
    mesh=plsc.VectorSubcoreMesh(core_axis_name="c", subcore_axis_name="s"),
    compiler_params=pltpu.CompilerParams(use_tc_tiling_on_sc=False,
                                         needs_layout_passes=False),
    scratch_types=[
        pltpu.VMEM((CHUNK * IN_DIM,), jnp.float32),
        pltpu.VMEM((CHUNK * DIMS[0],), jnp.float32),
        pltpu.VMEM((CHUNK * DIMS[1],), jnp.float32),
        pltpu.VMEM((TBLK,), jnp.int32),
        pltpu.VMEM((TBLK,), jnp.int32),
        pltpu.SemaphoreType.DMA,
        pltpu.SemaphoreType.DMA,
        pltpu.SemaphoreType.DMA,
    ],
)
def _sc_lcn(*refs):
    _sc_lcn_kernel(*refs)


def _pack_table(knn, w):
    wbits = lax.bitcast_convert_type(
        w.astype(jnp.bfloat16), jnp.uint16).astype(jnp.int32) << 16
    word = wbits | knn
    dim = knn.shape[0]
    nblk = dim // DBLK
    return word.T.reshape(K, nblk, DBLK).swapaxes(0, 1).reshape(-1)


def _fc_body(x_ref, wt_ref, b_ref, o_ref):
    o_ref[...] = jnp.dot(x_ref[...], wt_ref[...],
                         preferred_element_type=jnp.float32) + b_ref[...]


def _fc(x3, fc_wt, fc_b2):
    return pl.pallas_call(
        _fc_body,
        out_shape=jax.ShapeDtypeStruct((B, OUT_DIM), jnp.float32),
    )(x3, fc_wt, fc_b2)


def kernel(input, w0, w1, w2, fc_w, fc_b, knn0, knn1, knn2):
    tbl = jnp.concatenate([_pack_table(knn0, w0),
                           _pack_table(knn1, w1),
                           _pack_table(knn2, w2)])
    x3 = _sc_lcn(input.reshape(-1), tbl)
    return _fc(x3.reshape(B, DIMS[2]), fc_w.T,
               jnp.broadcast_to(fc_b, (1, OUT_DIM)))
